# baseline (speedup 1.0000x reference)
import functools

import jax
import jax.numpy as jnp
from jax import lax
from jax.experimental import pallas as pl
from jax.experimental.pallas import tpu as pltpu
from jax.experimental.pallas import tpu_sc as plsc

_NC, _NS, _L = 2, 16, 16
_NW = _NC * _NS
_B, _F, _D = 4096, 26, 64
_N = _B * _F
_G = 128
_NROW = _N // _G
_GPW = _NROW // _NW
_HALF = _GPW // 2


def _body(ids_hbm, offs_hbm, table_hbm, out_hbm, offs_v, idx_v, rows_v, sem):
    wid = lax.axis_index("s") * _NC + lax.axis_index("c")
    pltpu.sync_copy(offs_hbm.at[wid], offs_v)
    pltpu.sync_copy(ids_hbm.at[wid], idx_v)
    for g in range(_GPW):
        for i in range(_G // _L):
            sl = pl.ds(i * _L, _L)
            idx_v[g, sl] = idx_v[g, sl] + offs_v[g, sl]
    for h in range(2):
        base_row = wid * _GPW + h * _HALF
        cps = [
            pltpu.async_copy(
                table_hbm.at[idx_v.at[h * _HALF + g]], rows_v.at[g], sem
            )
            for g in range(_HALF)
        ]
        for cp in cps:
            cp.wait()
        pltpu.sync_copy(rows_v, out_hbm.at[pl.ds(base_row, _HALF)])


@jax.jit
def _sc_gather(ids, offs, table):
    mesh = plsc.VectorSubcoreMesh(core_axis_name="c", subcore_axis_name="s")
    f = pl.kernel(
        _body,
        out_type=jax.ShapeDtypeStruct((_NROW, _G, _D), jnp.float32),
        mesh=mesh,
        scratch_types=[
            pltpu.VMEM((_GPW, _G), jnp.int32),
            pltpu.VMEM((_GPW, _G), jnp.int32),
            pltpu.VMEM((_HALF, _G, _D), jnp.float32),
            pltpu.SemaphoreType.DMA,
        ],
        compiler_params=pltpu.CompilerParams(use_tc_tiling_on_sc=False),
    )
    return f(ids, offs, table)


def kernel(hash_ids, table, offsets):
    ids = hash_ids.astype(jnp.int32).reshape(_NW, _GPW, _G)
    offs = jnp.tile(offsets.astype(jnp.int32), _B).reshape(_NW, _GPW, _G)
    out = _sc_gather(ids, offs, table)
    return out.reshape(_B, _F, _D)


# speedup vs baseline: 2.3656x; 2.3656x over previous
import functools

import jax
import jax.numpy as jnp
from jax import lax
from jax.experimental import pallas as pl
from jax.experimental.pallas import tpu as pltpu
from jax.experimental.pallas import tpu_sc as plsc

_NC, _NS, _L = 2, 16, 16
_B, _F, _D = 4096, 26, 64
_RB = 100096
_HALF = _RB // 2
_FPC = _F // _NC
_UPC = _FPC * 8


def _body(ids_hbm, offs_hbm, tab_hbm, out_hbm,
          offs_v, idc_v, relh_v, val_v, row0, asm0, asm1,
          bnd0, part, sem):
    c = lax.axis_index("c")
    s = lax.axis_index("s")
    dl = lax.div(s, 2)
    h = lax.rem(s, 2)
    fbase = c * _FPC
    pltpu.sync_copy(offs_hbm, offs_v)
    row0[pl.ds(_HALF, _L)] = lax.full((_L,), 0.0, jnp.float32)

    def rb_of(off):
        return lax.bitwise_and(off, ~127)

    def off_at(f):
        return offs_v[0, pl.ds(f, _L)][0]

    def stage_src(f, a):
        off = off_at(f)
        rb = pl.multiple_of(rb_of(off), 128)
        a8 = pl.multiple_of(a * 8, 8)
        return tab_hbm.at[pl.ds(a8, 8), pl.ds(rb, _RB)]

    @pl.when(s == 0)
    def _():
        pltpu.async_copy(stage_src(fbase, 0), bnd0, sem)

    def u_step(u, carry):
        f = fbase + lax.div(u, 8)
        a = lax.rem(u, 8)

        @pl.when(a == 0)
        def _():
            pltpu.sync_copy(ids_hbm.at[f], idc_v)
            off = off_at(f)
            base = off - rb_of(off)
            for g in range(_B // _L):
                sl = pl.ds(g * _L, _L)
                rel = idc_v[0, sl] + base - h * _HALF
                inr = lax.lt(plsc.bitcast(rel, jnp.uint32),
                             lax.full((_L,), _HALF, jnp.uint32))
                relh_v[0, sl] = lax.select(
                    inr, rel, lax.full((_L,), _HALF, jnp.int32))

        @pl.when(s == 0)
        def _():
            pltpu.make_async_copy(stage_src(f, a), bnd0, sem).wait()
        plsc.subcore_barrier()
        pltpu.sync_copy(bnd0.at[dl, pl.ds(h * _HALF, _HALF)],
                        row0.at[pl.ds(0, _HALF)])
        plsc.subcore_barrier()

        @pl.when((s == 0) & (u < _UPC - 1))
        def _():
            pltpu.async_copy(
                stage_src(fbase + lax.div(u + 1, 8), lax.rem(u + 1, 8)),
                bnd0, sem)

        for g in range(_B // _L):
            sl = pl.ds(g * _L, _L)
            val_v[0, sl] = plsc.load_gather(row0, [relh_v[0, sl]])
        for t2 in range(_NS):
            pltpu.sync_copy(val_v.at[0, pl.ds(t2 * 256, 256)],
                            part.at[h, t2, dl])
        plsc.subcore_barrier()
        pltpu.sync_copy(part.at[0, s], asm0)
        pltpu.sync_copy(part.at[1, s], asm1)
        for r in range(8):
            for g in range(256 // _L):
                sl = pl.ds(g * _L, _L)
                asm0[r, sl] = asm0[r, sl] + asm1[r, sl]
        pltpu.sync_copy(
            asm0,
            out_hbm.at[f, pl.ds(pl.multiple_of(a * 8, 8), 8),
                       pl.ds(pl.multiple_of(s * 256, 256), 256)])
        return carry

    lax.fori_loop(0, _UPC, u_step, 0)


@jax.jit
def _sc_gather(ids_t, offs, tab_t):
    mesh = plsc.VectorSubcoreMesh(core_axis_name="c", subcore_axis_name="s")
    f = pl.kernel(
        _body,
        out_type=jax.ShapeDtypeStruct((_F, _D, _B), jnp.float32),
        mesh=mesh,
        scratch_types=[
            pltpu.VMEM((1, 48), jnp.int32),
            pltpu.VMEM((1, _B), jnp.int32),
            pltpu.VMEM((1, _B), jnp.int32),
            pltpu.VMEM((1, _B), jnp.float32),
            pltpu.VMEM((_HALF + _L,), jnp.float32),
            pltpu.VMEM((8, 256), jnp.float32),
            pltpu.VMEM((8, 256), jnp.float32),
            pltpu.VMEM_SHARED((8, _RB), jnp.float32),
            pltpu.VMEM_SHARED((2, _NS, 8, 256), jnp.float32),
            pltpu.SemaphoreType.DMA,
        ],
        compiler_params=pltpu.CompilerParams(needs_layout_passes=False),
    )
    return f(ids_t, offs, tab_t)


def kernel(hash_ids, table, offsets):
    ids_t = hash_ids.astype(jnp.int32).T.reshape(_F, 1, _B)
    offs = jnp.zeros((1, 48), jnp.int32).at[0, :_F].set(
        offsets.astype(jnp.int32))
    out = _sc_gather(ids_t, offs, table.T)
    return out.transpose(2, 0, 1)


# speedup vs baseline: 4.6562x; 1.9683x over previous
import functools

import jax
import jax.numpy as jnp
from jax import lax
from jax.experimental import pallas as pl
from jax.experimental.pallas import tpu as pltpu
from jax.experimental.pallas import tpu_sc as plsc

_NC, _NS, _L = 2, 16, 16
_NT = _NC * _NS
_B, _F, _D = 4096, 26, 64
_RB = 100096
_RPT = _D // _NT


def _body(ids_hbm, offs_hbm, tab_hbm, out_hbm,
          offs_v, idc_v, rel_v, val_v, row0):
    c = lax.axis_index("c")
    s = lax.axis_index("s")
    t = c * _NS + s
    pltpu.sync_copy(offs_hbm, offs_v)

    def f_step(f, carry):
        pltpu.sync_copy(ids_hbm.at[f], idc_v)
        off = offs_v[0, pl.ds(f, _L)][0]
        rb = pl.multiple_of(lax.bitwise_and(off, ~127), 128)
        base = off - rb
        for g in range(_B // _L):
            sl = pl.ds(g * _L, _L)
            rel_v[0, sl] = idc_v[0, sl] + base

        def d_step(j, carry2):
            d = t + j * _NT
            pltpu.sync_copy(tab_hbm.at[d, pl.ds(rb, _RB)], row0)
            for g in range(_B // _L):
                sl = pl.ds(g * _L, _L)
                val_v[0, sl] = plsc.load_gather(row0, [rel_v[0, sl]])
            pltpu.sync_copy(val_v.at[0], out_hbm.at[f, d])
            return carry2

        return lax.fori_loop(0, _RPT, d_step, carry)

    lax.fori_loop(0, _F, f_step, 0)


@jax.jit
def _sc_gather(ids_t, offs, tab_t):
    mesh = plsc.VectorSubcoreMesh(core_axis_name="c", subcore_axis_name="s")
    f = pl.kernel(
        _body,
        out_type=jax.ShapeDtypeStruct((_F, _D, _B), jnp.float32),
        mesh=mesh,
        scratch_types=[
            pltpu.VMEM((1, 48), jnp.int32),
            pltpu.VMEM((1, _B), jnp.int32),
            pltpu.VMEM((1, _B), jnp.int32),
            pltpu.VMEM((1, _B), jnp.float32),
            pltpu.VMEM((_RB,), jnp.float32),
        ],
        compiler_params=pltpu.CompilerParams(needs_layout_passes=False),
    )
    return f(ids_t, offs, tab_t)


def kernel(hash_ids, table, offsets):
    ids_t = hash_ids.astype(jnp.int32).T.reshape(_F, 1, _B)
    offs = jnp.zeros((1, 48), jnp.int32).at[0, :_F].set(
        offsets.astype(jnp.int32))
    out = _sc_gather(ids_t, offs, table.T)
    return out.transpose(2, 0, 1)
